# baseline (speedup 1.0000x reference)
import functools

import jax
import jax.numpy as jnp
from jax import lax
from jax.experimental import pallas as pl
from jax.experimental.pallas import tpu as pltpu
from jax.experimental.pallas import tpu_sc as plsc

_ROWS = 16384
_COLS = 4096
_NW = 32
_RPW = _ROWS // _NW
_RC = 8
_CHUNKS = _RPW // _RC
_WPR = _COLS // 4

_mesh = plsc.VectorSubcoreMesh(core_axis_name="c", subcore_axis_name="s")


@functools.partial(
    pl.kernel,
    out_type=[
        jax.ShapeDtypeStruct((_NW, 16), jnp.float32),
        jax.ShapeDtypeStruct((_NW, 16), jnp.float32),
    ],
    mesh=_mesh,
    compiler_params=pltpu.CompilerParams(needs_layout_passes=False),
    scratch_types=[
        pltpu.VMEM((_RC, _COLS), jnp.float32),
        pltpu.VMEM((_RC * _WPR,), jnp.int32),
        pltpu.VMEM((16,), jnp.float32),
        pltpu.VMEM((16,), jnp.float32),
    ],
)
def _sc_masked_sum(x_hbm, m_hbm, sum_hbm, cnt_hbm, xbuf, mbuf, sbuf, cbuf):
    wid = lax.axis_index("s") * 2 + lax.axis_index("c")
    row0 = wid * _RPW
    iota4 = lax.iota(jnp.int32, 16) * 4

    def chunk_body(k, carry):
        row = row0 + k * _RC
        pltpu.sync_copy(x_hbm.at[pl.ds(row, _RC)], xbuf)
        pltpu.sync_copy(m_hbm.at[pl.ds(row * _WPR, _RC * _WPR)], mbuf)
        for r in range(_RC):
            rowv = jnp.broadcast_to(jnp.int32(r), (16,))

            def g_body(g, carry, r=r, rowv=rowv):
                acc, cnt = carry
                mw = mbuf[pl.ds(r * _WPR + g * 16, 16)]
                base = g * 64 + iota4
                for b in range(4):
                    f = ((mw >> (8 * b)) & 1).astype(jnp.float32)
                    xb = plsc.load_gather(xbuf, [rowv, base + b])
                    acc = acc + xb * f
                    cnt = cnt + f
                return acc, cnt

            carry = lax.fori_loop(0, _WPR // 16, g_body, carry)
        return carry

    zero = jnp.zeros((16,), jnp.float32)
    acc, cnt = lax.fori_loop(0, _CHUNKS, chunk_body, (zero, zero))
    sbuf[...] = acc
    cbuf[...] = cnt
    pltpu.sync_copy(sbuf, sum_hbm.at[wid])
    pltpu.sync_copy(cbuf, cnt_hbm.at[wid])


def kernel(input, data_mask):
    m32 = data_mask.reshape(-1).view(jnp.int8).view(jnp.int32)
    s, c = _sc_masked_sum(input, m32)
    return jnp.sum(s) / jnp.sum(c)


# speedup vs baseline: 3.0105x; 3.0105x over previous
import functools

import jax
import jax.numpy as jnp
from jax import lax
from jax.experimental import pallas as pl
from jax.experimental.pallas import tpu as pltpu
from jax.experimental.pallas import tpu_sc as plsc

_ROWS = 16384
_COLS = 4096
_NW = 32
_RPW = _ROWS // _NW
_RC = 8
_CHUNKS = _RPW // _RC
_WPR = _COLS // 4

_mesh = plsc.VectorSubcoreMesh(core_axis_name="c", subcore_axis_name="s")


@functools.partial(
    pl.kernel,
    out_type=[
        jax.ShapeDtypeStruct((_NW, 16), jnp.float32),
        jax.ShapeDtypeStruct((_NW, 16), jnp.float32),
    ],
    mesh=_mesh,
    compiler_params=pltpu.CompilerParams(needs_layout_passes=False),
    scratch_types=[
        pltpu.VMEM((_RC, _COLS), jnp.float32),
        pltpu.VMEM((_RC, _COLS), jnp.float32),
        pltpu.VMEM((_RC * _WPR,), jnp.int32),
        pltpu.VMEM((_RC * _WPR,), jnp.int32),
        pltpu.VMEM((16,), jnp.float32),
        pltpu.VMEM((16,), jnp.float32),
        pltpu.SemaphoreType.DMA,
        pltpu.SemaphoreType.DMA,
        pltpu.SemaphoreType.DMA,
        pltpu.SemaphoreType.DMA,
    ],
)
def _sc_masked_sum(x_hbm, m_hbm, sum_hbm, cnt_hbm,
                   xbuf0, xbuf1, mbuf0, mbuf1, sbuf, cbuf,
                   semx0, semx1, semm0, semm1):
    wid = lax.axis_index("s") * 2 + lax.axis_index("c")
    row0 = wid * _RPW
    xbufs = (xbuf0, xbuf1)
    mbufs = (mbuf0, mbuf1)
    semxs = (semx0, semx1)
    semms = (semm0, semm1)

    def start_dma(kk, p):
        row = row0 + kk * _RC
        pltpu.async_copy(x_hbm.at[pl.ds(row, _RC)], xbufs[p], semxs[p])
        pltpu.async_copy(m_hbm.at[pl.ds(row * _WPR, _RC * _WPR)],
                         mbufs[p], semms[p])

    def wait_dma(kk, p):
        row = row0 + kk * _RC
        pltpu.make_async_copy(x_hbm.at[pl.ds(row, _RC)], xbufs[p],
                              semxs[p]).wait()
        pltpu.make_async_copy(m_hbm.at[pl.ds(row * _WPR, _RC * _WPR)],
                              mbufs[p], semms[p]).wait()

    def compute(xb, mb, carry):
        a0, a1, a2, a3, cnt32 = carry
        zero = jnp.zeros((16,), jnp.float32)
        for r in range(_RC):

            def g_body(gg, c, r=r):
                b0, b1, b2, b3, pc = c
                wbase = r * _WPR + gg * 64
                xbase = gg * 256
                for s in range(4):
                    mw = mb[pl.ds(wbase + s * 16, 16)]
                    xc = xbase + s * 64
                    b0 = b0 + jnp.where((mw & 0x1) != 0,
                                        xb[r, pl.ds(xc, 16)], 0.0)
                    b1 = b1 + jnp.where((mw & 0x100) != 0,
                                        xb[r, pl.ds(xc + 16, 16)], 0.0)
                    b2 = b2 + jnp.where((mw & 0x10000) != 0,
                                        xb[r, pl.ds(xc + 32, 16)], 0.0)
                    b3 = b3 + jnp.where((mw & 0x1000000) != 0,
                                        xb[r, pl.ds(xc + 48, 16)], 0.0)
                    pc = pc + (mw & 0x01010101)
                return b0, b1, b2, b3, pc

            a0, a1, a2, a3, pc = lax.fori_loop(
                0, _WPR // 64, g_body,
                (a0, a1, a2, a3, jnp.zeros((16,), jnp.int32)))
            cnt32 = (cnt32 + (pc & 0xFF) + ((pc >> 8) & 0xFF)
                     + ((pc >> 16) & 0xFF) + ((pc >> 24) & 0xFF))
        return a0, a1, a2, a3, cnt32

    for p in range(2):
        start_dma(p, p)

    def pair_body(k2, carry):
        for p in range(2):
            kk = k2 * 2 + p
            wait_dma(kk, p)
            carry = compute(xbufs[p], mbufs[p], carry)

            @pl.when(kk + 2 < _CHUNKS)
            def _(kk=kk, p=p):
                start_dma(kk + 2, p)

        return carry

    zero = jnp.zeros((16,), jnp.float32)
    zi = jnp.zeros((16,), jnp.int32)
    a0, a1, a2, a3, cnt32 = lax.fori_loop(
        0, _CHUNKS // 2, pair_body, (zero, zero, zero, zero, zi))
    sbuf[...] = a0 + a1 + a2 + a3
    cbuf[...] = cnt32.astype(jnp.float32)
    pltpu.sync_copy(sbuf, sum_hbm.at[wid])
    pltpu.sync_copy(cbuf, cnt_hbm.at[wid])


def kernel(input, data_mask):
    m_t = data_mask.reshape(-1, 4, 16).swapaxes(1, 2).reshape(-1)
    m32 = m_t.view(jnp.int8).view(jnp.int32)
    s, c = _sc_masked_sum(input, m32)
    return jnp.sum(s) / jnp.sum(c)


# speedup vs baseline: 59.0561x; 19.6167x over previous
import functools

import jax
import jax.numpy as jnp
from jax import lax
from jax.experimental import pallas as pl
from jax.experimental.pallas import tpu as pltpu
from jax.experimental.pallas import tpu_sc as plsc

_ROWS = 16384
_COLS = 4096
_NW = 32
_RPW = _ROWS // _NW
_RC = 8
_CHUNKS = _RPW // _RC
_WC = _COLS // 4

_mesh = plsc.VectorSubcoreMesh(core_axis_name="c", subcore_axis_name="s")


@functools.partial(
    pl.kernel,
    out_type=[
        jax.ShapeDtypeStruct((_NW, 16), jnp.float32),
        jax.ShapeDtypeStruct((_NW, 16), jnp.float32),
    ],
    mesh=_mesh,
    compiler_params=pltpu.CompilerParams(needs_layout_passes=False),
    scratch_types=[
        pltpu.VMEM((_RC, _COLS), jnp.float32),
        pltpu.VMEM((_RC, _COLS), jnp.float32),
        pltpu.VMEM((_RC // 4, _COLS), jnp.int32),
        pltpu.VMEM((_RC // 4, _COLS), jnp.int32),
        pltpu.VMEM((16,), jnp.float32),
        pltpu.VMEM((16,), jnp.float32),
        pltpu.SemaphoreType.DMA,
        pltpu.SemaphoreType.DMA,
        pltpu.SemaphoreType.DMA,
        pltpu.SemaphoreType.DMA,
    ],
)
def _sc_masked_sum(x_hbm, m_hbm, sum_hbm, cnt_hbm,
                   xbuf0, xbuf1, mbuf0, mbuf1, sbuf, cbuf,
                   semx0, semx1, semm0, semm1):
    wid = lax.axis_index("s") * 2 + lax.axis_index("c")
    row0 = wid * _RPW
    m32_hbm = m_hbm.bitcast(jnp.int32)
    xbufs = (xbuf0, xbuf1)
    mbufs = (mbuf0, mbuf1)
    semxs = (semx0, semx1)
    semms = (semm0, semm1)

    def start_dma(kk, p):
        row = row0 + kk * _RC
        wrow = pl.multiple_of(row // 4, 2)
        pltpu.async_copy(x_hbm.at[pl.ds(row, _RC)], xbufs[p], semxs[p])
        pltpu.async_copy(m32_hbm.at[pl.ds(wrow, _RC // 4)],
                         mbufs[p], semms[p])

    def wait_dma(kk, p):
        row = row0 + kk * _RC
        wrow = pl.multiple_of(row // 4, 2)
        pltpu.make_async_copy(x_hbm.at[pl.ds(row, _RC)], xbufs[p],
                              semxs[p]).wait()
        pltpu.make_async_copy(m32_hbm.at[pl.ds(wrow, _RC // 4)],
                              mbufs[p], semms[p]).wait()

    def compute(xb, mb, carry):
        a0, a1, a2, a3, cnt32 = carry
        for wr in range(_RC // 4):
            r4 = 4 * wr
            for half in range(2):

                def g_body(q, c, wr=wr, r4=r4):
                    b0, b1, b2, b3, pc = c
                    col = q * 16
                    mw = mb[wr, pl.ds(col, 16)]
                    b0 = b0 + jnp.where((mw & 0x1) != 0,
                                        xb[r4, pl.ds(col, 16)], 0.0)
                    b1 = b1 + jnp.where((mw & 0x100) != 0,
                                        xb[r4 + 1, pl.ds(col, 16)], 0.0)
                    b2 = b2 + jnp.where((mw & 0x10000) != 0,
                                        xb[r4 + 2, pl.ds(col, 16)], 0.0)
                    b3 = b3 + jnp.where((mw & 0x1000000) != 0,
                                        xb[r4 + 3, pl.ds(col, 16)], 0.0)
                    pc = pc + (mw & 0x01010101)
                    return b0, b1, b2, b3, pc

                a0, a1, a2, a3, pc = lax.fori_loop(
                    half * 128, half * 128 + 128, g_body,
                    (a0, a1, a2, a3, jnp.zeros((16,), jnp.int32)))
                cnt32 = (cnt32 + (pc & 0xFF) + ((pc >> 8) & 0xFF)
                         + ((pc >> 16) & 0xFF) + ((pc >> 24) & 0xFF))
        return a0, a1, a2, a3, cnt32

    for p in range(2):
        start_dma(p, p)

    def pair_body(k2, carry):
        for p in range(2):
            kk = k2 * 2 + p
            wait_dma(kk, p)
            carry = compute(xbufs[p], mbufs[p], carry)

            @pl.when(kk + 2 < _CHUNKS)
            def _(kk=kk, p=p):
                start_dma(kk + 2, p)

        return carry

    zero = jnp.zeros((16,), jnp.float32)
    zi = jnp.zeros((16,), jnp.int32)
    a0, a1, a2, a3, cnt32 = lax.fori_loop(
        0, _CHUNKS // 2, pair_body, (zero, zero, zero, zero, zi))
    sbuf[...] = a0 + a1 + a2 + a3
    cbuf[...] = cnt32.astype(jnp.float32)
    pltpu.sync_copy(sbuf, sum_hbm.at[wid])
    pltpu.sync_copy(cbuf, cnt_hbm.at[wid])


def kernel(input, data_mask):
    s, c = _sc_masked_sum(input, data_mask.view(jnp.int8))
    return jnp.sum(s) / jnp.sum(c)
